# Initial kernel scaffold; baseline (speedup 1.0000x reference)
import jax
import jax.numpy as jnp
from jax.experimental import pallas as pl


def kernel(inputs, Y, W, E):
    raise NotImplementedError("write your pallas kernel here")


# speedup vs baseline: 111.6898x; 111.6898x over previous
import jax
import jax.numpy as jnp
from jax import lax
from jax.experimental import pallas as pl
from jax.experimental.pallas import tpu as pltpu
from jax.experimental.pallas import tpu_sc as plsc

K = 16
N = 64
M = 128
B = 64
_ROWS = B * (N + 1)
_ROWS_PAD = 4224
_LANE_BLK = 384
_NEG = -3.0e38
_LN2 = 0.6931471805599453


def _two_sum(a, b):
    s = a + b
    bb = s - a
    return s, (a - (s - bb)) + (b - bb)


def _ds_norm(h, l):
    s = h + l
    return s, l - (s - h)


def _ds_add(xh, xl, yh, yl):
    s, e = _two_sum(xh, yh)
    e = e + (xl + yl)
    return _ds_norm(s, e)


def _mask12(x):
    xi = lax.bitcast_convert_type(x, jnp.uint32)
    return lax.bitcast_convert_type(xi & jnp.uint32(0xFFFFF000), jnp.float32)


def _lexmax(mh, ml, ch, cl):
    take = (ch > mh) | ((ch == mh) & (cl > ml))
    return jnp.where(take, ch, mh), jnp.where(take, cl, ml)


def _u_body(xT_ref, wh_ref, wl_ref, uh_ref, ul_ref):
    x = xT_ref[...]
    xh = _mask12(x)
    xl = x - xh
    for j in range(K):
        wh = wh_ref[:, j:j + 1]
        wl = wl_ref[:, j:j + 1]
        whh = _mask12(wh)
        whl = wh - whh
        h = xh * whh
        l = (xh * whl + xl * wh) + x * wl
        r = M
        while r > 1:
            half = r // 2
            h, l = _ds_add(h[:half], l[:half], h[half:], l[half:])
            r = half
        uh_ref[j:j + 1, :] = h
        ul_ref[j:j + 1, :] = l


def _u_pallas(xT, Wh, Wl):
    grid = _ROWS_PAD // _LANE_BLK
    return pl.pallas_call(
        _u_body,
        grid=(grid,),
        in_specs=[
            pl.BlockSpec((M, _LANE_BLK), lambda c: (c - c, c)),
            pl.BlockSpec((M, K), lambda c: (c - c, c - c)),
            pl.BlockSpec((M, K), lambda c: (c - c, c - c)),
        ],
        out_specs=[
            pl.BlockSpec((K, _LANE_BLK), lambda c: (c - c, c)),
            pl.BlockSpec((K, _LANE_BLK), lambda c: (c - c, c)),
        ],
        out_shape=[
            jax.ShapeDtypeStruct((K, _ROWS_PAD), jnp.float32),
            jax.ShapeDtypeStruct((K, _ROWS_PAD), jnp.float32),
        ],
    )(xT, Wh, Wl)


def _loss_body(fp_ref, out_ref):
    fp = fp_ref[...]
    F = fp[:, 0:1]
    ps = fp[:, 1:2]
    ea = fp[:, 2:3]
    logZ = jnp.log(ps) + jnp.float32(_LN2) * ea
    out_ref[0, 0] = jnp.sum(logZ - F) * jnp.float32(1.0 / B)


def _loss_pallas(fpack):
    return pl.pallas_call(
        _loss_body,
        out_shape=jax.ShapeDtypeStruct((1, 1), jnp.float32),
        out_specs=pl.BlockSpec(memory_space=pltpu.SMEM),
    )(fpack)


def _sc_body(uh_hbm, ul_hbm, y_hbm, eh_hbm, el_hbm, eth_hbm, etl_hbm,
             expe_hbm, ef_hbm,
             yhat_hbm, fpack_hbm,
             uh_v, ul_v, y_v, eh_v, el_v, eth_v, etl_v, expe_v, ef_v,
             hh_v, hl_v, yh_v, fp_v):
    i32 = jnp.int32
    wid = lax.axis_index("s").astype(i32) * i32(2) + lax.axis_index("c").astype(i32)

    pltpu.sync_copy(eh_hbm, eh_v)
    pltpu.sync_copy(el_hbm, el_v)
    pltpu.sync_copy(eth_hbm, eth_v)
    pltpu.sync_copy(etl_hbm, etl_v)
    pltpu.sync_copy(expe_hbm, expe_v)
    pltpu.sync_copy(ef_hbm, ef_v)

    iota = lax.iota(jnp.int32, 16)
    zero16f = jnp.zeros((16,), jnp.float32)
    neg16 = jnp.full((16,), jnp.float32(_NEG))

    gdn = lax.GatherDimensionNumbers(offset_dims=(), collapsed_slice_dims=(0,),
                                     start_index_map=(0,))

    def _perm(x, k):
        idx = (iota ^ i32(k)).reshape(16, 1)
        return lax.gather(x, idx, gdn, slice_sizes=(1,),
                          mode=lax.GatherScatterMode.PROMISE_IN_BOUNDS)

    def _allreduce(x, op):
        for k in (1, 2, 4, 8):
            x = op(x, _perm(x, k))
        return x

    def _argmax_ds(th, tl):
        mhv = _allreduce(th, jnp.maximum)
        msk = th == mhv
        lm = jnp.where(msk, tl, neg16)
        m2v = _allreduce(lm, jnp.maximum)
        msk2 = msk & (lm == m2v)
        idxv = _allreduce(jnp.where(msk2, iota, jnp.full((16,), i32(99))),
                          jnp.minimum)
        return idxv[0]

    for ex in range(B // 32):
        b = wid * i32(B // 32) + i32(ex)
        pltpu.sync_copy(uh_hbm.at[b], uh_v)
        pltpu.sync_copy(ul_hbm.at[b], ul_v)
        pltpu.sync_copy(y_hbm.at[b], y_v)

        hh_v[0] = uh_v[1]
        hl_v[0] = ul_v[1]

        def _fwd(it):
            t = it + i32(2)
            prow_h = hh_v[it]
            prow_l = hl_v[it]
            mh = neg16
            ml = neg16
            for i in range(K):
                bh = jnp.full((16,), prow_h[i])
                bl = jnp.full((16,), prow_l[i])
                ch, cl = _ds_add(bh, bl, eh_v[i], el_v[i])
                mh, ml = _lexmax(mh, ml, ch, cl)
            ph, plo = _ds_add(mh, ml, uh_v[t], ul_v[t])
            hh_v[it + i32(1)] = ph
            hl_v[it + i32(1)] = plo
            return it + i32(1)

        lax.while_loop(lambda it: it < i32(N - 1), _fwd, i32(0))

        for c in range(4):
            yh_v[pl.ds(c * 16, 16)] = jnp.zeros((16,), jnp.int32)
        lb0 = _argmax_ds(hh_v[N - 1], hl_v[N - 1]) + i32(1)
        tail = jnp.where(iota == i32(0), jnp.full((16,), lb0),
                         jnp.where(iota == i32(1),
                                   jnp.full((16,), i32(K + 1)),
                                   jnp.zeros((16,), i32)))
        yh_v[pl.ds(N, 16)] = tail
        lane0 = iota == i32(0)

        def _bwd(carry):
            t, lb = carry
            th, tl = _ds_add(hh_v[t - i32(1)], hl_v[t - i32(1)],
                             eth_v[lb - i32(1)], etl_v[lb - i32(1)])
            nb = _argmax_ds(th, tl) + i32(1)
            plsc.store_scatter(yh_v, [jnp.full((16,), t)],
                               jnp.full((16,), nb), mask=lane0)
            return t - i32(1), nb

        lax.while_loop(lambda c: c[0] >= i32(1), _bwd, (i32(N - 1), lb0))

        phi0 = jnp.exp(uh_v[1])

        def _zfwd(carry):
            phi, eacc, t = carry
            acc = zero16f
            for i in range(K):
                acc = acc + jnp.full((16,), phi[i]) * expe_v[i]
            phi = jnp.exp(uh_v[t]) * acc
            mxv = _allreduce(phi, jnp.maximum)
            ebits = lax.bitcast_convert_type(mxv, jnp.int32)
            ev = ((ebits >> i32(23)) & i32(0xFF)) - i32(127)
            scale = lax.bitcast_convert_type((i32(127) - ev) << i32(23),
                                             jnp.float32)
            return phi * scale, eacc + ev, t + i32(1)

        phiN, eaccN, _t = lax.while_loop(
            lambda c: c[2] <= i32(N), _zfwd,
            (phi0, jnp.zeros((16,), jnp.int32), i32(2)))
        phi_sumv = _allreduce(phiN, lambda a, b: a + b)
        e_accv = eaccN

        facc = zero16f
        for c in range(4):
            tvec = iota + i32(c * 16 + 1)
            y = plsc.load_gather(y_v, [tvec])
            um = (y >= i32(1)) & (y <= i32(16))
            yc = jnp.minimum(jnp.maximum(y - i32(1), i32(0)), i32(15))
            uv = plsc.load_gather(uh_v, [tvec, yc])
            facc = facc + jnp.where(um, uv, zero16f)
            em = tvec <= i32(63)
            ya = y
            yb = plsc.load_gather(y_v, [tvec + i32(1)])
            ev = plsc.load_gather(ef_v, [ya, yb])
            facc = facc + jnp.where(em, ev, zero16f)
        Fv = _allreduce(facc, lambda a, b: a + b)

        fp_v[...] = jnp.where(
            iota == i32(0), Fv,
            jnp.where(iota == i32(1), phi_sumv,
                      jnp.where(iota == i32(2), e_accv.astype(jnp.float32),
                                zero16f)))

        pltpu.sync_copy(yh_v, yhat_hbm.at[b])
        pltpu.sync_copy(fp_v, fpack_hbm.at[b])


def _sc_pallas(Uh, Ul, Ypad, Eh, El, ETh, ETl, expE, Efull):
    mesh = plsc.VectorSubcoreMesh(core_axis_name="c", subcore_axis_name="s")
    f32 = jnp.float32
    kern = pl.kernel(
        _sc_body,
        mesh=mesh,
        compiler_params=pltpu.CompilerParams(needs_layout_passes=False),
        out_type=[
            jax.ShapeDtypeStruct((B, 80), jnp.int32),
            jax.ShapeDtypeStruct((B, 16), jnp.float32),
        ],
        scratch_types=[
            pltpu.VMEM((N + 1, K), f32),
            pltpu.VMEM((N + 1, K), f32),
            pltpu.VMEM((80,), jnp.int32),
            pltpu.VMEM((K, K), f32),
            pltpu.VMEM((K, K), f32),
            pltpu.VMEM((K, K), f32),
            pltpu.VMEM((K, K), f32),
            pltpu.VMEM((K, K), f32),
            pltpu.VMEM((K + 2, K + 2), f32),
            pltpu.VMEM((N, K), f32),
            pltpu.VMEM((N, K), f32),
            pltpu.VMEM((80,), jnp.int32),
            pltpu.VMEM((K,), f32),
        ],
    )
    return kern(Uh, Ul, Ypad, Eh, El, ETh, ETl, expE, Efull)


def kernel(inputs, Y, W, E):
    f32 = jnp.float32
    f64 = jnp.float64
    Wc = W[:, 1:K + 1]
    Wh = Wc.astype(f32)
    Wl = (Wc - Wh.astype(f64)).astype(f32)
    Ecore = E[1:-1, 1:-1]
    Eh = Ecore.astype(f32)
    El = (Ecore - Eh.astype(f64)).astype(f32)
    EcT = Ecore.T
    ETh = EcT.astype(f32)
    ETl = (EcT - ETh.astype(f64)).astype(f32)
    expE = jnp.exp(Ecore).astype(f32)
    Efull = E.astype(f32)

    x = inputs.reshape(_ROWS, M)
    xT = jnp.zeros((M, _ROWS_PAD), f32).at[:, :_ROWS].set(x.T)
    UhT, UlT = _u_pallas(xT, Wh, Wl)
    Uh = UhT[:, :_ROWS].T.reshape(B, N + 1, K)
    Ul = UlT[:, :_ROWS].T.reshape(B, N + 1, K)

    Ypad = jnp.zeros((B, 80), jnp.int32).at[:, :N + 2].set(Y.astype(jnp.int32))

    yhat_pad, fpack = _sc_pallas(Uh, Ul, Ypad, Eh, El, ETh, ETl, expE, Efull)
    loss32 = _loss_pallas(fpack)

    Y_hat = yhat_pad[:, :N + 2]
    loss = loss32[0, 0].astype(f64)
    return (Y_hat, loss)


# speedup vs baseline: 119.6831x; 1.0716x over previous
import jax
import jax.numpy as jnp
from jax import lax
from jax.experimental import pallas as pl
from jax.experimental.pallas import tpu as pltpu
from jax.experimental.pallas import tpu_sc as plsc

K = 16
N = 64
M = 128
B = 64
_ROWS = B * (N + 1)
_ROW_BLK = 416
_NEG = -3.0e38
_LN2 = 0.6931471805599453


def _two_sum(a, b):
    s = a + b
    bb = s - a
    return s, (a - (s - bb)) + (b - bb)


def _ds_norm(h, l):
    s = h + l
    return s, l - (s - h)


def _ds_add(xh, xl, yh, yl):
    s, e = _two_sum(xh, yh)
    e = e + (xl + yl)
    return _ds_norm(s, e)


def _mask12(x):
    xi = lax.bitcast_convert_type(x, jnp.uint32)
    return lax.bitcast_convert_type(xi & jnp.uint32(0xFFFFF000), jnp.float32)


def _lexmax(mh, ml, ch, cl):
    take = (ch > mh) | ((ch == mh) & (cl > ml))
    return jnp.where(take, ch, mh), jnp.where(take, cl, ml)


def _u_body(x_ref, wh_ref, wl_ref, uh_ref, ul_ref):
    x = x_ref[...].T
    xh = _mask12(x)
    xl = x - xh
    hs = []
    ls = []
    for j in range(K):
        wh = wh_ref[:, j:j + 1]
        wl = wl_ref[:, j:j + 1]
        whh = _mask12(wh)
        whl = wh - whh
        h = xh * whh
        l = (xh * whl + xl * wh) + x * wl
        r = M
        while r > 1:
            half = r // 2
            h, l = _ds_add(h[:half], l[:half], h[half:], l[half:])
            r = half
        hs.append(h)
        ls.append(l)
    uh_ref[...] = jnp.concatenate(hs, axis=0).T
    ul_ref[...] = jnp.concatenate(ls, axis=0).T


def _u_pallas(x, Wh, Wl):
    grid = _ROWS // _ROW_BLK
    return pl.pallas_call(
        _u_body,
        grid=(grid,),
        in_specs=[
            pl.BlockSpec((_ROW_BLK, M), lambda c: (c, c - c)),
            pl.BlockSpec((M, K), lambda c: (c - c, c - c)),
            pl.BlockSpec((M, K), lambda c: (c - c, c - c)),
        ],
        out_specs=[
            pl.BlockSpec((_ROW_BLK, K), lambda c: (c, c - c)),
            pl.BlockSpec((_ROW_BLK, K), lambda c: (c, c - c)),
        ],
        out_shape=[
            jax.ShapeDtypeStruct((_ROWS, K), jnp.float32),
            jax.ShapeDtypeStruct((_ROWS, K), jnp.float32),
        ],
    )(x, Wh, Wl)


def _loss_body(fp_ref, out_ref):
    fp = fp_ref[...]
    F = fp[:, 0:1]
    ps = fp[:, 1:2]
    ea = fp[:, 2:3]
    logZ = jnp.log(ps) + jnp.float32(_LN2) * ea
    out_ref[0, 0] = jnp.sum(logZ - F) * jnp.float32(1.0 / B)


def _loss_pallas(fpack):
    return pl.pallas_call(
        _loss_body,
        out_shape=jax.ShapeDtypeStruct((1, 1), jnp.float32),
        out_specs=pl.BlockSpec(memory_space=pltpu.SMEM),
    )(fpack)


def _sc_body(uh_hbm, ul_hbm, y_hbm, eh_hbm, el_hbm, eth_hbm, etl_hbm,
             expe_hbm, ef_hbm,
             yhat_hbm, fpack_hbm,
             uh_v, ul_v, y_v, eh_v, el_v, eth_v, etl_v, expe_v, ef_v,
             hh_v, hl_v, yh_v, fp_v):
    i32 = jnp.int32
    wid = lax.axis_index("s").astype(i32) * i32(2) + lax.axis_index("c").astype(i32)

    pltpu.sync_copy(eh_hbm, eh_v)
    pltpu.sync_copy(el_hbm, el_v)
    pltpu.sync_copy(eth_hbm, eth_v)
    pltpu.sync_copy(etl_hbm, etl_v)
    pltpu.sync_copy(expe_hbm, expe_v)
    pltpu.sync_copy(ef_hbm, ef_v)

    iota = lax.iota(jnp.int32, 16)
    zero16f = jnp.zeros((16,), jnp.float32)
    neg16 = jnp.full((16,), jnp.float32(_NEG))

    gdn = lax.GatherDimensionNumbers(offset_dims=(), collapsed_slice_dims=(0,),
                                     start_index_map=(0,))
    _SPLAT_IDX = [jnp.full((16, 1), i, jnp.int32) for i in range(K)]

    def _perm(x, k):
        idx = (iota ^ i32(k)).reshape(16, 1)
        return lax.gather(x, idx, gdn, slice_sizes=(1,),
                          mode=lax.GatherScatterMode.PROMISE_IN_BOUNDS)

    def _allreduce(x, op):
        for k in (1, 2, 4, 8):
            x = op(x, _perm(x, k))
        return x

    def _argmax_ds(th, tl):
        mhv = _allreduce(th, jnp.maximum)
        msk = th == mhv
        lm = jnp.where(msk, tl, neg16)
        m2v = _allreduce(lm, jnp.maximum)
        msk2 = msk & (lm == m2v)
        idxv = _allreduce(jnp.where(msk2, iota, jnp.full((16,), i32(99))),
                          jnp.minimum)
        return idxv[0]

    for ex in range(B // 32):
        b = wid * i32(B // 32) + i32(ex)
        pltpu.sync_copy(uh_hbm.at[b], uh_v)
        pltpu.sync_copy(ul_hbm.at[b], ul_v)
        pltpu.sync_copy(y_hbm.at[b], y_v)

        hh_v[0] = uh_v[1]
        hl_v[0] = ul_v[1]
        uh1 = uh_v[1]
        phi0 = jnp.exp(uh1)

        def _splat(v, i):
            return lax.gather(v, _SPLAT_IDX[i], gdn, slice_sizes=(1,),
                              mode=lax.GatherScatterMode.PROMISE_IN_BOUNDS)

        def _fwd(carry):
            it, phi, eacc = carry
            t = it + i32(2)
            uh_t = uh_v[t]
            ul_t = ul_v[t]
            prow_h = hh_v[it]
            prow_l = hl_v[it]
            mh = neg16
            ml = neg16
            acc = zero16f
            for i in range(K):
                bh = _splat(prow_h, i)
                bl = _splat(prow_l, i)
                ch, cl = _ds_add(bh, bl, eh_v[i], el_v[i])
                mh, ml = _lexmax(mh, ml, ch, cl)
                acc = acc + _splat(phi, i) * expe_v[i]
            ph, plo = _ds_add(mh, ml, uh_t, ul_t)
            hh_v[it + i32(1)] = ph
            hl_v[it + i32(1)] = plo
            phi = jnp.exp(uh_t) * acc
            mxv = _allreduce(phi, jnp.maximum)
            ebits = lax.bitcast_convert_type(mxv, jnp.int32)
            ev = ((ebits >> i32(23)) & i32(0xFF)) - i32(127)
            scale = lax.bitcast_convert_type((i32(127) - ev) << i32(23),
                                             jnp.float32)
            return it + i32(1), phi * scale, eacc + ev

        _it, phiN, eaccN = lax.while_loop(
            lambda c: c[0] < i32(N - 1), _fwd,
            (i32(0), phi0, jnp.zeros((16,), jnp.int32)))
        phi_sumv = _allreduce(phiN, lambda a, b: a + b)
        e_accv = eaccN

        for c in range(4):
            yh_v[pl.ds(c * 16, 16)] = jnp.zeros((16,), jnp.int32)
        lb0 = _argmax_ds(hh_v[N - 1], hl_v[N - 1]) + i32(1)
        tail = jnp.where(iota == i32(0), jnp.full((16,), lb0),
                         jnp.where(iota == i32(1),
                                   jnp.full((16,), i32(K + 1)),
                                   jnp.zeros((16,), i32)))
        yh_v[pl.ds(N, 16)] = tail
        lane0 = iota == i32(0)

        def _bwd(carry):
            t, lb = carry
            th, tl = _ds_add(hh_v[t - i32(1)], hl_v[t - i32(1)],
                             eth_v[lb - i32(1)], etl_v[lb - i32(1)])
            nb = _argmax_ds(th, tl) + i32(1)
            plsc.store_scatter(yh_v, [jnp.full((16,), t)],
                               jnp.full((16,), nb), mask=lane0)
            return t - i32(1), nb

        lax.while_loop(lambda c: c[0] >= i32(1), _bwd, (i32(N - 1), lb0))

        facc = zero16f
        for c in range(4):
            tvec = iota + i32(c * 16 + 1)
            y = plsc.load_gather(y_v, [tvec])
            um = (y >= i32(1)) & (y <= i32(16))
            yc = jnp.minimum(jnp.maximum(y - i32(1), i32(0)), i32(15))
            uv = plsc.load_gather(uh_v, [tvec, yc])
            facc = facc + jnp.where(um, uv, zero16f)
            em = tvec <= i32(63)
            ya = y
            yb = plsc.load_gather(y_v, [tvec + i32(1)])
            ev = plsc.load_gather(ef_v, [ya, yb])
            facc = facc + jnp.where(em, ev, zero16f)
        Fv = _allreduce(facc, lambda a, b: a + b)

        fp_v[...] = jnp.where(
            iota == i32(0), Fv,
            jnp.where(iota == i32(1), phi_sumv,
                      jnp.where(iota == i32(2), e_accv.astype(jnp.float32),
                                zero16f)))

        pltpu.sync_copy(yh_v, yhat_hbm.at[b])
        pltpu.sync_copy(fp_v, fpack_hbm.at[b])


def _sc_pallas(Uh, Ul, Ypad, Eh, El, ETh, ETl, expE, Efull):
    mesh = plsc.VectorSubcoreMesh(core_axis_name="c", subcore_axis_name="s")
    f32 = jnp.float32
    kern = pl.kernel(
        _sc_body,
        mesh=mesh,
        compiler_params=pltpu.CompilerParams(needs_layout_passes=False),
        out_type=[
            jax.ShapeDtypeStruct((B, 80), jnp.int32),
            jax.ShapeDtypeStruct((B, 16), jnp.float32),
        ],
        scratch_types=[
            pltpu.VMEM((N + 1, K), f32),
            pltpu.VMEM((N + 1, K), f32),
            pltpu.VMEM((80,), jnp.int32),
            pltpu.VMEM((K, K), f32),
            pltpu.VMEM((K, K), f32),
            pltpu.VMEM((K, K), f32),
            pltpu.VMEM((K, K), f32),
            pltpu.VMEM((K, K), f32),
            pltpu.VMEM((K + 2, K + 2), f32),
            pltpu.VMEM((N, K), f32),
            pltpu.VMEM((N, K), f32),
            pltpu.VMEM((80,), jnp.int32),
            pltpu.VMEM((K,), f32),
        ],
    )
    return kern(Uh, Ul, Ypad, Eh, El, ETh, ETl, expE, Efull)


def kernel(inputs, Y, W, E):
    f32 = jnp.float32
    f64 = jnp.float64
    Wc = W[:, 1:K + 1]
    Wh = Wc.astype(f32)
    Wl = (Wc - Wh.astype(f64)).astype(f32)
    Ecore = E[1:-1, 1:-1]
    Eh = Ecore.astype(f32)
    El = (Ecore - Eh.astype(f64)).astype(f32)
    EcT = Ecore.T
    ETh = EcT.astype(f32)
    ETl = (EcT - ETh.astype(f64)).astype(f32)
    expE = jnp.exp(Ecore).astype(f32)
    Efull = E.astype(f32)

    x = inputs.reshape(_ROWS, M)
    Uh2, Ul2 = _u_pallas(x, Wh, Wl)
    Uh = Uh2.reshape(B, N + 1, K)
    Ul = Ul2.reshape(B, N + 1, K)

    Ypad = jnp.zeros((B, 80), jnp.int32).at[:, :N + 2].set(Y.astype(jnp.int32))

    yhat_pad, fpack = _sc_pallas(Uh, Ul, Ypad, Eh, El, ETh, ETl, expE, Efull)
    loss32 = _loss_pallas(fpack)

    Y_hat = yhat_pad[:, :N + 2]
    loss = loss32[0, 0].astype(f64)
    return (Y_hat, loss)


# speedup vs baseline: 120.2681x; 1.0049x over previous
import jax
import jax.numpy as jnp
from jax import lax
from jax.experimental import pallas as pl
from jax.experimental.pallas import tpu as pltpu
from jax.experimental.pallas import tpu_sc as plsc

K = 16
N = 64
M = 128
B = 64
_ROWS = B * (N + 1)
_ROW_BLK = 416
_NEG = -3.0e38
_LN2 = 0.6931471805599453


def _two_sum(a, b):
    s = a + b
    bb = s - a
    return s, (a - (s - bb)) + (b - bb)


def _ds_norm(h, l):
    s = h + l
    return s, l - (s - h)


def _ds_add(xh, xl, yh, yl):
    s, e = _two_sum(xh, yh)
    e = e + (xl + yl)
    return _ds_norm(s, e)


def _mask12(x):
    xi = lax.bitcast_convert_type(x, jnp.uint32)
    return lax.bitcast_convert_type(xi & jnp.uint32(0xFFFFF000), jnp.float32)


def _lexmax(mh, ml, ch, cl):
    take = (ch > mh) | ((ch == mh) & (cl > ml))
    return jnp.where(take, ch, mh), jnp.where(take, cl, ml)


def _u_body(x_ref, wh_ref, wl_ref, uh_ref, ul_ref):
    x = x_ref[...].T
    xh = _mask12(x)
    xl = x - xh
    hs = []
    ls = []
    for j in range(K):
        wh = wh_ref[:, j:j + 1]
        wl = wl_ref[:, j:j + 1]
        whh = _mask12(wh)
        whl = wh - whh
        h = xh * whh
        l = (xh * whl + xl * wh) + x * wl
        r = M
        while r > 1:
            half = r // 2
            h, l = _ds_add(h[:half], l[:half], h[half:], l[half:])
            r = half
        hs.append(h)
        ls.append(l)
    uh_ref[...] = jnp.concatenate(hs, axis=0).T
    ul_ref[...] = jnp.concatenate(ls, axis=0).T


def _u_pallas(x, Wh, Wl):
    grid = _ROWS // _ROW_BLK
    return pl.pallas_call(
        _u_body,
        grid=(grid,),
        in_specs=[
            pl.BlockSpec((_ROW_BLK, M), lambda c: (c, c - c)),
            pl.BlockSpec((M, K), lambda c: (c - c, c - c)),
            pl.BlockSpec((M, K), lambda c: (c - c, c - c)),
        ],
        out_specs=[
            pl.BlockSpec((_ROW_BLK, K), lambda c: (c, c - c)),
            pl.BlockSpec((_ROW_BLK, K), lambda c: (c, c - c)),
        ],
        out_shape=[
            jax.ShapeDtypeStruct((_ROWS, K), jnp.float32),
            jax.ShapeDtypeStruct((_ROWS, K), jnp.float32),
        ],
    )(x, Wh, Wl)


def _loss_body(fp_ref, out_ref):
    fp = fp_ref[...]
    F = fp[:, 0:1]
    ps = fp[:, 1:2]
    ea = fp[:, 2:3]
    logZ = jnp.log(ps) + jnp.float32(_LN2) * ea
    out_ref[0, 0] = jnp.sum(logZ - F) * jnp.float32(1.0 / B)


def _loss_pallas(fpack):
    return pl.pallas_call(
        _loss_body,
        out_shape=jax.ShapeDtypeStruct((1, 1), jnp.float32),
        out_specs=pl.BlockSpec(memory_space=pltpu.SMEM),
    )(fpack)


def _sc_body(uh_hbm, ul_hbm, y_hbm, eh_hbm, el_hbm, eth_hbm, etl_hbm,
             expe_hbm, ef_hbm,
             yhat_hbm, fpack_hbm,
             uh0_v, ul0_v, uh1_v, ul1_v, y0_v, y1_v,
             eh_v, el_v, eth_v, etl_v, expe_v, ef_v,
             hh0_v, hl0_v, hh1_v, hl1_v, yh_v, fp_v):
    i32 = jnp.int32
    wid = lax.axis_index("s").astype(i32) * i32(2) + lax.axis_index("c").astype(i32)

    pltpu.sync_copy(eh_hbm, eh_v)
    pltpu.sync_copy(el_hbm, el_v)
    pltpu.sync_copy(eth_hbm, eth_v)
    pltpu.sync_copy(etl_hbm, etl_v)
    pltpu.sync_copy(expe_hbm, expe_v)
    pltpu.sync_copy(ef_hbm, ef_v)

    iota = lax.iota(jnp.int32, 16)
    zero16f = jnp.zeros((16,), jnp.float32)
    neg16 = jnp.full((16,), jnp.float32(_NEG))

    gdn = lax.GatherDimensionNumbers(offset_dims=(), collapsed_slice_dims=(0,),
                                     start_index_map=(0,))
    _SPLAT_IDX = [jnp.full((16, 1), i, jnp.int32) for i in range(K)]

    def _perm(x, k):
        idx = (iota ^ i32(k)).reshape(16, 1)
        return lax.gather(x, idx, gdn, slice_sizes=(1,),
                          mode=lax.GatherScatterMode.PROMISE_IN_BOUNDS)

    def _allreduce(x, op):
        for k in (1, 2, 4, 8):
            x = op(x, _perm(x, k))
        return x

    def _argmax_ds(th, tl):
        mhv = _allreduce(th, jnp.maximum)
        msk = th == mhv
        lm = jnp.where(msk, tl, neg16)
        m2v = _allreduce(lm, jnp.maximum)
        msk2 = msk & (lm == m2v)
        idxv = _allreduce(jnp.where(msk2, iota, jnp.full((16,), i32(99))),
                          jnp.minimum)
        return idxv[0]

    def _splat(v, i):
        return lax.gather(v, _SPLAT_IDX[i], gdn, slice_sizes=(1,),
                          mode=lax.GatherScatterMode.PROMISE_IN_BOUNDS)

    b0 = wid * i32(2)
    b1 = b0 + i32(1)
    pltpu.sync_copy(uh_hbm.at[b0], uh0_v)
    pltpu.sync_copy(ul_hbm.at[b0], ul0_v)
    pltpu.sync_copy(y_hbm.at[b0], y0_v)
    pltpu.sync_copy(uh_hbm.at[b1], uh1_v)
    pltpu.sync_copy(ul_hbm.at[b1], ul1_v)
    pltpu.sync_copy(y_hbm.at[b1], y1_v)

    uh = (uh0_v, uh1_v)
    ul = (ul0_v, ul1_v)
    hh = (hh0_v, hh1_v)
    hl = (hl0_v, hl1_v)
    yv = (y0_v, y1_v)

    for e in range(2):
        hh[e][0] = uh[e][1]
        hl[e][0] = ul[e][1]
    phi_init = (jnp.exp(uh[0][1]), jnp.exp(uh[1][1]))

    def _fwd(carry):
        it, phi_a, phi_b, ea_a, ea_b = carry
        t = it + i32(2)
        phis = (phi_a, phi_b)
        outs = []
        for e in range(2):
            uh_t = uh[e][t]
            ul_t = ul[e][t]
            prow_h = hh[e][it]
            prow_l = hl[e][it]
            phi = phis[e]
            cands = []
            terms = []
            for i in range(K):
                bh = _splat(prow_h, i)
                bl = _splat(prow_l, i)
                cands.append(_ds_add(bh, bl, eh_v[i], el_v[i]))
                terms.append(_splat(phi, i) * expe_v[i])
            while len(cands) > 1:
                cands = [_lexmax(a[0], a[1], c[0], c[1])
                         for a, c in zip(cands[::2], cands[1::2])]
            while len(terms) > 1:
                terms = [a + c for a, c in zip(terms[::2], terms[1::2])]
            mh, ml = cands[0]
            ph, plo = _ds_add(mh, ml, uh_t, ul_t)
            hh[e][it + i32(1)] = ph
            hl[e][it + i32(1)] = plo
            phi = jnp.exp(uh_t) * terms[0]
            mxv = _allreduce(phi, jnp.maximum)
            ebits = lax.bitcast_convert_type(mxv, jnp.int32)
            ev = ((ebits >> i32(23)) & i32(0xFF)) - i32(127)
            scale = lax.bitcast_convert_type((i32(127) - ev) << i32(23),
                                             jnp.float32)
            outs.append((phi * scale, ev))
        return (it + i32(1), outs[0][0], outs[1][0],
                ea_a + outs[0][1], ea_b + outs[1][1])

    _it, phiN0, phiN1, eaccN0, eaccN1 = lax.while_loop(
        lambda c: c[0] < i32(N - 1), _fwd,
        (i32(0), phi_init[0], phi_init[1],
         jnp.zeros((16,), jnp.int32), jnp.zeros((16,), jnp.int32)))
    phiN = (phiN0, phiN1)
    eaccN = (eaccN0, eaccN1)

    lane0 = iota == i32(0)
    for e in range(2):
        b = b0 if e == 0 else b1
        for c in range(4):
            yh_v[pl.ds(c * 16, 16)] = jnp.zeros((16,), jnp.int32)
        lb0 = _argmax_ds(hh[e][N - 1], hl[e][N - 1]) + i32(1)
        tail = jnp.where(iota == i32(0), jnp.full((16,), lb0),
                         jnp.where(iota == i32(1),
                                   jnp.full((16,), i32(K + 1)),
                                   jnp.zeros((16,), i32)))
        yh_v[pl.ds(N, 16)] = tail

        hh_e = hh[e]
        hl_e = hl[e]

        def _bwd(carry):
            t, lb = carry
            th, tl = _ds_add(hh_e[t - i32(1)], hl_e[t - i32(1)],
                             eth_v[lb - i32(1)], etl_v[lb - i32(1)])
            nb = _argmax_ds(th, tl) + i32(1)
            plsc.store_scatter(yh_v, [jnp.full((16,), t)],
                               jnp.full((16,), nb), mask=lane0)
            return t - i32(1), nb

        lax.while_loop(lambda c: c[0] >= i32(1), _bwd, (i32(N - 1), lb0))

        phi_sumv = _allreduce(phiN[e], lambda a, c: a + c)
        e_accv = eaccN[e]

        facc = zero16f
        for c in range(4):
            tvec = iota + i32(c * 16 + 1)
            y = plsc.load_gather(yv[e], [tvec])
            um = (y >= i32(1)) & (y <= i32(16))
            yc = jnp.minimum(jnp.maximum(y - i32(1), i32(0)), i32(15))
            uv = plsc.load_gather(uh[e], [tvec, yc])
            facc = facc + jnp.where(um, uv, zero16f)
            em = tvec <= i32(63)
            ya = y
            yb = plsc.load_gather(yv[e], [tvec + i32(1)])
            ev = plsc.load_gather(ef_v, [ya, yb])
            facc = facc + jnp.where(em, ev, zero16f)
        Fv = _allreduce(facc, lambda a, c: a + c)

        fp_v[...] = jnp.where(
            iota == i32(0), Fv,
            jnp.where(iota == i32(1), phi_sumv,
                      jnp.where(iota == i32(2), e_accv.astype(jnp.float32),
                                zero16f)))

        pltpu.sync_copy(yh_v, yhat_hbm.at[b])
        pltpu.sync_copy(fp_v, fpack_hbm.at[b])


def _sc_pallas(Uh, Ul, Ypad, Eh, El, ETh, ETl, expE, Efull):
    mesh = plsc.VectorSubcoreMesh(core_axis_name="c", subcore_axis_name="s")
    f32 = jnp.float32
    kern = pl.kernel(
        _sc_body,
        mesh=mesh,
        compiler_params=pltpu.CompilerParams(needs_layout_passes=False),
        out_type=[
            jax.ShapeDtypeStruct((B, 80), jnp.int32),
            jax.ShapeDtypeStruct((B, 16), jnp.float32),
        ],
        scratch_types=[
            pltpu.VMEM((N + 1, K), f32),
            pltpu.VMEM((N + 1, K), f32),
            pltpu.VMEM((N + 1, K), f32),
            pltpu.VMEM((N + 1, K), f32),
            pltpu.VMEM((80,), jnp.int32),
            pltpu.VMEM((80,), jnp.int32),
            pltpu.VMEM((K, K), f32),
            pltpu.VMEM((K, K), f32),
            pltpu.VMEM((K, K), f32),
            pltpu.VMEM((K, K), f32),
            pltpu.VMEM((K, K), f32),
            pltpu.VMEM((K + 2, K + 2), f32),
            pltpu.VMEM((N, K), f32),
            pltpu.VMEM((N, K), f32),
            pltpu.VMEM((N, K), f32),
            pltpu.VMEM((N, K), f32),
            pltpu.VMEM((80,), jnp.int32),
            pltpu.VMEM((K,), f32),
        ],
    )
    return kern(Uh, Ul, Ypad, Eh, El, ETh, ETl, expE, Efull)


def kernel(inputs, Y, W, E):
    f32 = jnp.float32
    f64 = jnp.float64
    Wc = W[:, 1:K + 1]
    Wh = Wc.astype(f32)
    Wl = (Wc - Wh.astype(f64)).astype(f32)
    Ecore = E[1:-1, 1:-1]
    Eh = Ecore.astype(f32)
    El = (Ecore - Eh.astype(f64)).astype(f32)
    EcT = Ecore.T
    ETh = EcT.astype(f32)
    ETl = (EcT - ETh.astype(f64)).astype(f32)
    expE = jnp.exp(Ecore).astype(f32)
    Efull = E.astype(f32)

    x = inputs.reshape(_ROWS, M)
    Uh2, Ul2 = _u_pallas(x, Wh, Wl)
    Uh = Uh2.reshape(B, N + 1, K)
    Ul = Ul2.reshape(B, N + 1, K)

    Ypad = jnp.zeros((B, 80), jnp.int32).at[:, :N + 2].set(Y.astype(jnp.int32))

    yhat_pad, fpack = _sc_pallas(Uh, Ul, Ypad, Eh, El, ETh, ETl, expE, Efull)
    loss32 = _loss_pallas(fpack)

    Y_hat = yhat_pad[:, :N + 2]
    loss = loss32[0, 0].astype(f64)
    return (Y_hat, loss)


# speedup vs baseline: 120.3158x; 1.0004x over previous
import jax
import jax.numpy as jnp
from jax import lax
from jax.experimental import pallas as pl
from jax.experimental.pallas import tpu as pltpu
from jax.experimental.pallas import tpu_sc as plsc

K = 16
N = 64
M = 128
B = 64
_ROWS = B * (N + 1)
_ROW_BLK = 416
_NEG = -3.0e38
_LN2 = 0.6931471805599453


def _two_sum(a, b):
    s = a + b
    bb = s - a
    return s, (a - (s - bb)) + (b - bb)


def _ds_norm(h, l):
    s = h + l
    return s, l - (s - h)


def _ds_add(xh, xl, yh, yl):
    s, e = _two_sum(xh, yh)
    e = e + (xl + yl)
    return _ds_norm(s, e)


def _mask12(x):
    xi = lax.bitcast_convert_type(x, jnp.uint32)
    return lax.bitcast_convert_type(xi & jnp.uint32(0xFFFFF000), jnp.float32)


def _lexmax(mh, ml, ch, cl):
    take = (ch > mh) | ((ch == mh) & (cl > ml))
    return jnp.where(take, ch, mh), jnp.where(take, cl, ml)


def _u_body(x_ref, wh_ref, wl_ref, uh_ref, ul_ref):
    x = x_ref[...].T
    xh = _mask12(x)
    xl = x - xh
    hs = []
    ls = []
    for j in range(K):
        wh = wh_ref[:, j:j + 1]
        wl = wl_ref[:, j:j + 1]
        whh = _mask12(wh)
        whl = wh - whh
        h = xh * whh
        l = (xh * whl + xl * wh) + x * wl
        r = M
        while r > 1:
            half = r // 2
            h, l = _ds_add(h[:half], l[:half], h[half:], l[half:])
            r = half
        hs.append(h)
        ls.append(l)
    uh_ref[...] = jnp.concatenate(hs, axis=0).T
    ul_ref[...] = jnp.concatenate(ls, axis=0).T


def _u_pallas(x, Wh, Wl):
    grid = _ROWS // _ROW_BLK
    return pl.pallas_call(
        _u_body,
        grid=(grid,),
        in_specs=[
            pl.BlockSpec((_ROW_BLK, M), lambda c: (c, c - c)),
            pl.BlockSpec((M, K), lambda c: (c - c, c - c)),
            pl.BlockSpec((M, K), lambda c: (c - c, c - c)),
        ],
        out_specs=[
            pl.BlockSpec((_ROW_BLK, K), lambda c: (c, c - c)),
            pl.BlockSpec((_ROW_BLK, K), lambda c: (c, c - c)),
        ],
        out_shape=[
            jax.ShapeDtypeStruct((_ROWS, K), jnp.float32),
            jax.ShapeDtypeStruct((_ROWS, K), jnp.float32),
        ],
    )(x, Wh, Wl)


def _loss_body(fp_ref, out_ref):
    fp = fp_ref[...]
    F = fp[:, 0:1]
    ps = fp[:, 1:2]
    ea = fp[:, 2:3]
    logZ = jnp.log(ps) + jnp.float32(_LN2) * ea
    out_ref[0, 0] = jnp.sum(logZ - F) * jnp.float32(1.0 / B)


def _loss_pallas(fpack):
    return pl.pallas_call(
        _loss_body,
        out_shape=jax.ShapeDtypeStruct((1, 1), jnp.float32),
        out_specs=pl.BlockSpec(memory_space=pltpu.SMEM),
    )(fpack)


def _sc_body(uh_hbm, ul_hbm, y_hbm, eh_hbm, el_hbm, eth_hbm, etl_hbm,
             expe_hbm, ef_hbm,
             yhat_hbm, fpack_hbm,
             uh0_v, ul0_v, uh1_v, ul1_v, y0_v, y1_v,
             eh_v, el_v, eth_v, etl_v, expe_v, ef_v,
             hh0_v, hl0_v, hh1_v, hl1_v, yh_v, fp_v):
    i32 = jnp.int32
    wid = lax.axis_index("s").astype(i32) * i32(2) + lax.axis_index("c").astype(i32)

    pltpu.sync_copy(eh_hbm, eh_v)
    pltpu.sync_copy(el_hbm, el_v)
    pltpu.sync_copy(eth_hbm, eth_v)
    pltpu.sync_copy(etl_hbm, etl_v)
    pltpu.sync_copy(expe_hbm, expe_v)
    pltpu.sync_copy(ef_hbm, ef_v)

    iota = lax.iota(jnp.int32, 16)
    zero16f = jnp.zeros((16,), jnp.float32)
    neg16 = jnp.full((16,), jnp.float32(_NEG))

    gdn = lax.GatherDimensionNumbers(offset_dims=(), collapsed_slice_dims=(0,),
                                     start_index_map=(0,))
    _SPLAT_IDX = [jnp.full((16, 1), i, jnp.int32) for i in range(K)]

    def _perm(x, k):
        idx = (iota ^ i32(k)).reshape(16, 1)
        return lax.gather(x, idx, gdn, slice_sizes=(1,),
                          mode=lax.GatherScatterMode.PROMISE_IN_BOUNDS)

    def _allreduce(x, op):
        for k in (1, 2, 4, 8):
            x = op(x, _perm(x, k))
        return x

    def _argmax_ds(th, tl):
        mhv = _allreduce(th, jnp.maximum)
        msk = th == mhv
        lm = jnp.where(msk, tl, neg16)
        m2v = _allreduce(lm, jnp.maximum)
        msk2 = msk & (lm == m2v)
        idxv = _allreduce(jnp.where(msk2, iota, jnp.full((16,), i32(99))),
                          jnp.minimum)
        return idxv[0]

    def _splat(v, i):
        return lax.gather(v, _SPLAT_IDX[i], gdn, slice_sizes=(1,),
                          mode=lax.GatherScatterMode.PROMISE_IN_BOUNDS)

    b0 = wid * i32(2)
    b1 = b0 + i32(1)
    pltpu.sync_copy(uh_hbm.at[b0], uh0_v)
    pltpu.sync_copy(ul_hbm.at[b0], ul0_v)
    pltpu.sync_copy(y_hbm.at[b0], y0_v)
    pltpu.sync_copy(uh_hbm.at[b1], uh1_v)
    pltpu.sync_copy(ul_hbm.at[b1], ul1_v)
    pltpu.sync_copy(y_hbm.at[b1], y1_v)

    uh = (uh0_v, uh1_v)
    ul = (ul0_v, ul1_v)
    hh = (hh0_v, hh1_v)
    hl = (hl0_v, hl1_v)
    yv = (y0_v, y1_v)

    pi_init = []
    for e in range(2):
        hh[e][0] = uh[e][1]
        hl[e][0] = ul[e][1]
        pi_init += [uh[e][1], ul[e][1]]
    phi_init = (jnp.exp(pi_init[0]), jnp.exp(pi_init[2]))

    def _fwd(carry):
        it, pih_a, pil_a, pih_b, pil_b, phi_a, phi_b, ea_a, ea_b = carry
        t = it + i32(2)
        pis = ((pih_a, pil_a), (pih_b, pil_b))
        phis = (phi_a, phi_b)
        outs = []
        for e in range(2):
            uh_t = uh[e][t]
            ul_t = ul[e][t]
            prow_h, prow_l = pis[e]
            phi = phis[e]
            cands = []
            terms = []
            for i in range(K):
                bh = _splat(prow_h, i)
                bl = _splat(prow_l, i)
                cands.append(_ds_add(bh, bl, eh_v[i], el_v[i]))
                terms.append(_splat(phi, i) * expe_v[i])
            while len(cands) > 1:
                cands = [_lexmax(a[0], a[1], c[0], c[1])
                         for a, c in zip(cands[::2], cands[1::2])]
            while len(terms) > 1:
                terms = [a + c for a, c in zip(terms[::2], terms[1::2])]
            mh, ml = cands[0]
            ph, plo = _ds_add(mh, ml, uh_t, ul_t)
            hh[e][it + i32(1)] = ph
            hl[e][it + i32(1)] = plo
            phi = jnp.exp(uh_t) * terms[0]
            mxv = _allreduce(phi, jnp.maximum)
            ebits = lax.bitcast_convert_type(mxv, jnp.int32)
            ev = ((ebits >> i32(23)) & i32(0xFF)) - i32(127)
            scale = lax.bitcast_convert_type((i32(127) - ev) << i32(23),
                                             jnp.float32)
            outs.append((ph, plo, phi * scale, ev))
        return (it + i32(1), outs[0][0], outs[0][1], outs[1][0], outs[1][1],
                outs[0][2], outs[1][2], ea_a + outs[0][3], ea_b + outs[1][3])

    _fin = lax.while_loop(
        lambda c: c[0] < i32(N - 1), _fwd,
        (i32(0), pi_init[0], pi_init[1], pi_init[2], pi_init[3],
         phi_init[0], phi_init[1],
         jnp.zeros((16,), jnp.int32), jnp.zeros((16,), jnp.int32)))
    phiN = (_fin[5], _fin[6])
    eaccN = (_fin[7], _fin[8])

    lane0 = iota == i32(0)
    for e in range(2):
        b = b0 if e == 0 else b1
        for c in range(4):
            yh_v[pl.ds(c * 16, 16)] = jnp.zeros((16,), jnp.int32)
        lb0 = _argmax_ds(hh[e][N - 1], hl[e][N - 1]) + i32(1)
        tail = jnp.where(iota == i32(0), jnp.full((16,), lb0),
                         jnp.where(iota == i32(1),
                                   jnp.full((16,), i32(K + 1)),
                                   jnp.zeros((16,), i32)))
        yh_v[pl.ds(N, 16)] = tail

        hh_e = hh[e]
        hl_e = hl[e]

        def _bwd(carry):
            t, lb = carry
            th, tl = _ds_add(hh_e[t - i32(1)], hl_e[t - i32(1)],
                             eth_v[lb - i32(1)], etl_v[lb - i32(1)])
            nb = _argmax_ds(th, tl) + i32(1)
            plsc.store_scatter(yh_v, [jnp.full((16,), t)],
                               jnp.full((16,), nb), mask=lane0)
            return t - i32(1), nb

        lax.while_loop(lambda c: c[0] >= i32(1), _bwd, (i32(N - 1), lb0))

        phi_sumv = _allreduce(phiN[e], lambda a, c: a + c)
        e_accv = eaccN[e]

        facc = zero16f
        for c in range(4):
            tvec = iota + i32(c * 16 + 1)
            y = plsc.load_gather(yv[e], [tvec])
            um = (y >= i32(1)) & (y <= i32(16))
            yc = jnp.minimum(jnp.maximum(y - i32(1), i32(0)), i32(15))
            uv = plsc.load_gather(uh[e], [tvec, yc])
            facc = facc + jnp.where(um, uv, zero16f)
            em = tvec <= i32(63)
            ya = y
            yb = plsc.load_gather(yv[e], [tvec + i32(1)])
            ev = plsc.load_gather(ef_v, [ya, yb])
            facc = facc + jnp.where(em, ev, zero16f)
        Fv = _allreduce(facc, lambda a, c: a + c)

        fp_v[...] = jnp.where(
            iota == i32(0), Fv,
            jnp.where(iota == i32(1), phi_sumv,
                      jnp.where(iota == i32(2), e_accv.astype(jnp.float32),
                                zero16f)))

        pltpu.sync_copy(yh_v, yhat_hbm.at[b])
        pltpu.sync_copy(fp_v, fpack_hbm.at[b])


def _sc_pallas(Uh, Ul, Ypad, Eh, El, ETh, ETl, expE, Efull):
    mesh = plsc.VectorSubcoreMesh(core_axis_name="c", subcore_axis_name="s")
    f32 = jnp.float32
    kern = pl.kernel(
        _sc_body,
        mesh=mesh,
        compiler_params=pltpu.CompilerParams(needs_layout_passes=False),
        out_type=[
            jax.ShapeDtypeStruct((B, 80), jnp.int32),
            jax.ShapeDtypeStruct((B, 16), jnp.float32),
        ],
        scratch_types=[
            pltpu.VMEM((N + 1, K), f32),
            pltpu.VMEM((N + 1, K), f32),
            pltpu.VMEM((N + 1, K), f32),
            pltpu.VMEM((N + 1, K), f32),
            pltpu.VMEM((80,), jnp.int32),
            pltpu.VMEM((80,), jnp.int32),
            pltpu.VMEM((K, K), f32),
            pltpu.VMEM((K, K), f32),
            pltpu.VMEM((K, K), f32),
            pltpu.VMEM((K, K), f32),
            pltpu.VMEM((K, K), f32),
            pltpu.VMEM((K + 2, K + 2), f32),
            pltpu.VMEM((N, K), f32),
            pltpu.VMEM((N, K), f32),
            pltpu.VMEM((N, K), f32),
            pltpu.VMEM((N, K), f32),
            pltpu.VMEM((80,), jnp.int32),
            pltpu.VMEM((K,), f32),
        ],
    )
    return kern(Uh, Ul, Ypad, Eh, El, ETh, ETl, expE, Efull)


def kernel(inputs, Y, W, E):
    f32 = jnp.float32
    f64 = jnp.float64
    Wc = W[:, 1:K + 1]
    Wh = Wc.astype(f32)
    Wl = (Wc - Wh.astype(f64)).astype(f32)
    Ecore = E[1:-1, 1:-1]
    Eh = Ecore.astype(f32)
    El = (Ecore - Eh.astype(f64)).astype(f32)
    EcT = Ecore.T
    ETh = EcT.astype(f32)
    ETl = (EcT - ETh.astype(f64)).astype(f32)
    expE = jnp.exp(Ecore).astype(f32)
    Efull = E.astype(f32)

    x = inputs.reshape(_ROWS, M)
    Uh2, Ul2 = _u_pallas(x, Wh, Wl)
    Uh = Uh2.reshape(B, N + 1, K)
    Ul = Ul2.reshape(B, N + 1, K)

    Ypad = jnp.zeros((B, 80), jnp.int32).at[:, :N + 2].set(Y.astype(jnp.int32))

    yhat_pad, fpack = _sc_pallas(Uh, Ul, Ypad, Eh, El, ETh, ETl, expE, Efull)
    loss32 = _loss_pallas(fpack)

    Y_hat = yhat_pad[:, :N + 2]
    loss = loss32[0, 0].astype(f64)
    return (Y_hat, loss)


# speedup vs baseline: 127.5496x; 1.0601x over previous
import jax
import jax.numpy as jnp
from jax import lax
from jax.experimental import pallas as pl
from jax.experimental.pallas import tpu as pltpu
from jax.experimental.pallas import tpu_sc as plsc

K = 16
N = 64
M = 128
B = 64
_ROWS = B * (N + 1)
_ROW_BLK = 416
_NEG = -3.0e38
_LN2 = 0.6931471805599453


def _two_sum(a, b):
    s = a + b
    bb = s - a
    return s, (a - (s - bb)) + (b - bb)


def _ds_norm(h, l):
    s = h + l
    return s, l - (s - h)


def _ds_add(xh, xl, yh, yl):
    s, e = _two_sum(xh, yh)
    e = e + (xl + yl)
    return _ds_norm(s, e)


def _mask12(x):
    xi = lax.bitcast_convert_type(x, jnp.uint32)
    return lax.bitcast_convert_type(xi & jnp.uint32(0xFFFFF000), jnp.float32)


def _lexmax(mh, ml, ch, cl):
    take = (ch > mh) | ((ch == mh) & (cl > ml))
    return jnp.where(take, ch, mh), jnp.where(take, cl, ml)


def _u_body(x_ref, wh_ref, wl_ref, uh_ref, ul_ref):
    x = x_ref[...].T
    xh = _mask12(x)
    xl = x - xh
    hs = []
    ls = []
    for j in range(K):
        wh = wh_ref[:, j:j + 1]
        wl = wl_ref[:, j:j + 1]
        whh = _mask12(wh)
        whl = wh - whh
        h = xh * whh
        l = (xh * whl + xl * wh) + x * wl
        r = M
        while r > 1:
            half = r // 2
            h, l = _ds_add(h[:half], l[:half], h[half:], l[half:])
            r = half
        hs.append(h)
        ls.append(l)
    uh_ref[...] = jnp.concatenate(hs, axis=0).T
    ul_ref[...] = jnp.concatenate(ls, axis=0).T


def _u_pallas(x, Wh, Wl):
    grid = _ROWS // _ROW_BLK
    return pl.pallas_call(
        _u_body,
        grid=(grid,),
        in_specs=[
            pl.BlockSpec((_ROW_BLK, M), lambda c: (c, c - c)),
            pl.BlockSpec((M, K), lambda c: (c - c, c - c)),
            pl.BlockSpec((M, K), lambda c: (c - c, c - c)),
        ],
        out_specs=[
            pl.BlockSpec((_ROW_BLK, K), lambda c: (c, c - c)),
            pl.BlockSpec((_ROW_BLK, K), lambda c: (c, c - c)),
        ],
        out_shape=[
            jax.ShapeDtypeStruct((_ROWS, K), jnp.float32),
            jax.ShapeDtypeStruct((_ROWS, K), jnp.float32),
        ],
    )(x, Wh, Wl)


def _loss_body(fp_ref, out_ref):
    fp = fp_ref[...]
    F = fp[:, 0:1]
    ps = fp[:, 1:2]
    ea = fp[:, 2:3]
    logZ = jnp.log(ps) + jnp.float32(_LN2) * ea
    out_ref[0, 0] = jnp.sum(logZ - F) * jnp.float32(1.0 / B)


def _loss_pallas(fpack):
    return pl.pallas_call(
        _loss_body,
        out_shape=jax.ShapeDtypeStruct((1, 1), jnp.float32),
        out_specs=pl.BlockSpec(memory_space=pltpu.SMEM),
    )(fpack)


def _sc_body(uh_hbm, ul_hbm, y_hbm, eh_hbm, el_hbm, eth_hbm, etl_hbm,
             expe_hbm, ef_hbm,
             yhat_hbm, fpack_hbm,
             uh0_v, ul0_v, uh1_v, ul1_v, y0_v, y1_v,
             eh_v, el_v, eth_v, etl_v, expe_v, ef_v,
             hh0_v, hl0_v, hh1_v, hl1_v, yh_v, fp_v, dma_sem):
    i32 = jnp.int32
    wid = lax.axis_index("s").astype(i32) * i32(2) + lax.axis_index("c").astype(i32)


    iota = lax.iota(jnp.int32, 16)
    zero16f = jnp.zeros((16,), jnp.float32)
    neg16 = jnp.full((16,), jnp.float32(_NEG))

    gdn = lax.GatherDimensionNumbers(offset_dims=(), collapsed_slice_dims=(0,),
                                     start_index_map=(0,))
    _SPLAT_IDX = [jnp.full((16, 1), i, jnp.int32) for i in range(K)]

    def _perm(x, k):
        idx = (iota ^ i32(k)).reshape(16, 1)
        return lax.gather(x, idx, gdn, slice_sizes=(1,),
                          mode=lax.GatherScatterMode.PROMISE_IN_BOUNDS)

    def _allreduce(x, op):
        for k in (1, 2, 4, 8):
            x = op(x, _perm(x, k))
        return x

    def _argmax_ds(th, tl):
        mhv = _allreduce(th, jnp.maximum)
        msk = th == mhv
        lm = jnp.where(msk, tl, neg16)
        m2v = _allreduce(lm, jnp.maximum)
        msk2 = msk & (lm == m2v)
        idxv = _allreduce(jnp.where(msk2, iota, jnp.full((16,), i32(99))),
                          jnp.minimum)
        return idxv[0]

    def _splat(v, i):
        return lax.gather(v, _SPLAT_IDX[i], gdn, slice_sizes=(1,),
                          mode=lax.GatherScatterMode.PROMISE_IN_BOUNDS)

    b0 = wid * i32(2)
    b1 = b0 + i32(1)
    _copies = [
        pltpu.async_copy(eh_hbm, eh_v, dma_sem),
        pltpu.async_copy(el_hbm, el_v, dma_sem),
        pltpu.async_copy(eth_hbm, eth_v, dma_sem),
        pltpu.async_copy(etl_hbm, etl_v, dma_sem),
        pltpu.async_copy(expe_hbm, expe_v, dma_sem),
        pltpu.async_copy(ef_hbm, ef_v, dma_sem),
        pltpu.async_copy(uh_hbm.at[b0], uh0_v, dma_sem),
        pltpu.async_copy(ul_hbm.at[b0], ul0_v, dma_sem),
        pltpu.async_copy(y_hbm.at[b0], y0_v, dma_sem),
        pltpu.async_copy(uh_hbm.at[b1], uh1_v, dma_sem),
        pltpu.async_copy(ul_hbm.at[b1], ul1_v, dma_sem),
        pltpu.async_copy(y_hbm.at[b1], y1_v, dma_sem),
    ]
    for _c in _copies:
        _c.wait()

    uh = (uh0_v, uh1_v)
    ul = (ul0_v, ul1_v)
    hh = (hh0_v, hh1_v)
    hl = (hl0_v, hl1_v)
    yv = (y0_v, y1_v)

    pi_init = []
    for e in range(2):
        hh[e][0] = uh[e][1]
        hl[e][0] = ul[e][1]
        pi_init += [uh[e][1], ul[e][1]]
    phi_init = (jnp.exp(pi_init[0]), jnp.exp(pi_init[2]))

    def _fwd(carry):
        it, pih_a, pil_a, pih_b, pil_b, phi_a, phi_b, ea_a, ea_b = carry
        t = it + i32(2)
        pis = ((pih_a, pil_a), (pih_b, pil_b))
        phis = (phi_a, phi_b)
        outs = []
        for e in range(2):
            uh_t = uh[e][t]
            ul_t = ul[e][t]
            prow_h, prow_l = pis[e]
            phi = phis[e]
            cands = []
            terms = []
            for i in range(K):
                bh = _splat(prow_h, i)
                bl = _splat(prow_l, i)
                cands.append(_ds_add(bh, bl, eh_v[i], el_v[i]))
                terms.append(_splat(phi, i) * expe_v[i])
            while len(cands) > 1:
                cands = [_lexmax(a[0], a[1], c[0], c[1])
                         for a, c in zip(cands[::2], cands[1::2])]
            while len(terms) > 1:
                terms = [a + c for a, c in zip(terms[::2], terms[1::2])]
            mh, ml = cands[0]
            ph, plo = _ds_add(mh, ml, uh_t, ul_t)
            hh[e][it + i32(1)] = ph
            hl[e][it + i32(1)] = plo
            phi = jnp.exp(uh_t) * terms[0]
            mxv = _allreduce(phi, jnp.maximum)
            ebits = lax.bitcast_convert_type(mxv, jnp.int32)
            ev = ((ebits >> i32(23)) & i32(0xFF)) - i32(127)
            scale = lax.bitcast_convert_type((i32(127) - ev) << i32(23),
                                             jnp.float32)
            outs.append((ph, plo, phi * scale, ev))
        return (it + i32(1), outs[0][0], outs[0][1], outs[1][0], outs[1][1],
                outs[0][2], outs[1][2], ea_a + outs[0][3], ea_b + outs[1][3])

    _fin = lax.while_loop(
        lambda c: c[0] < i32(N - 1), _fwd,
        (i32(0), pi_init[0], pi_init[1], pi_init[2], pi_init[3],
         phi_init[0], phi_init[1],
         jnp.zeros((16,), jnp.int32), jnp.zeros((16,), jnp.int32)))
    phiN = (_fin[5], _fin[6])
    eaccN = (_fin[7], _fin[8])

    lane0 = iota == i32(0)
    for e in range(2):
        b = b0 if e == 0 else b1
        for c in range(4):
            yh_v[pl.ds(c * 16, 16)] = jnp.zeros((16,), jnp.int32)
        lb0 = _argmax_ds(hh[e][N - 1], hl[e][N - 1]) + i32(1)
        tail = jnp.where(iota == i32(0), jnp.full((16,), lb0),
                         jnp.where(iota == i32(1),
                                   jnp.full((16,), i32(K + 1)),
                                   jnp.zeros((16,), i32)))
        yh_v[pl.ds(N, 16)] = tail

        hh_e = hh[e]
        hl_e = hl[e]

        def _bwd(carry):
            t, lb = carry
            th, tl = _ds_add(hh_e[t - i32(1)], hl_e[t - i32(1)],
                             eth_v[lb - i32(1)], etl_v[lb - i32(1)])
            nb = _argmax_ds(th, tl) + i32(1)
            plsc.store_scatter(yh_v, [jnp.full((16,), t)],
                               jnp.full((16,), nb), mask=lane0)
            return t - i32(1), nb

        lax.while_loop(lambda c: c[0] >= i32(1), _bwd, (i32(N - 1), lb0))

        phi_sumv = _allreduce(phiN[e], lambda a, c: a + c)
        e_accv = eaccN[e]

        facc = zero16f
        for c in range(4):
            tvec = iota + i32(c * 16 + 1)
            y = plsc.load_gather(yv[e], [tvec])
            um = (y >= i32(1)) & (y <= i32(16))
            yc = jnp.minimum(jnp.maximum(y - i32(1), i32(0)), i32(15))
            uv = plsc.load_gather(uh[e], [tvec, yc])
            facc = facc + jnp.where(um, uv, zero16f)
            em = tvec <= i32(63)
            ya = y
            yb = plsc.load_gather(yv[e], [tvec + i32(1)])
            ev = plsc.load_gather(ef_v, [ya, yb])
            facc = facc + jnp.where(em, ev, zero16f)
        Fv = _allreduce(facc, lambda a, c: a + c)

        fp_v[...] = jnp.where(
            iota == i32(0), Fv,
            jnp.where(iota == i32(1), phi_sumv,
                      jnp.where(iota == i32(2), e_accv.astype(jnp.float32),
                                zero16f)))

        pltpu.sync_copy(yh_v, yhat_hbm.at[b])
        pltpu.sync_copy(fp_v, fpack_hbm.at[b])


def _sc_pallas(Uh, Ul, Ypad, Eh, El, ETh, ETl, expE, Efull):
    mesh = plsc.VectorSubcoreMesh(core_axis_name="c", subcore_axis_name="s")
    f32 = jnp.float32
    kern = pl.kernel(
        _sc_body,
        mesh=mesh,
        compiler_params=pltpu.CompilerParams(needs_layout_passes=False),
        out_type=[
            jax.ShapeDtypeStruct((B, 80), jnp.int32),
            jax.ShapeDtypeStruct((B, 16), jnp.float32),
        ],
        scratch_types=[
            pltpu.VMEM((N + 1, K), f32),
            pltpu.VMEM((N + 1, K), f32),
            pltpu.VMEM((N + 1, K), f32),
            pltpu.VMEM((N + 1, K), f32),
            pltpu.VMEM((80,), jnp.int32),
            pltpu.VMEM((80,), jnp.int32),
            pltpu.VMEM((K, K), f32),
            pltpu.VMEM((K, K), f32),
            pltpu.VMEM((K, K), f32),
            pltpu.VMEM((K, K), f32),
            pltpu.VMEM((K, K), f32),
            pltpu.VMEM((K + 2, K + 2), f32),
            pltpu.VMEM((N, K), f32),
            pltpu.VMEM((N, K), f32),
            pltpu.VMEM((N, K), f32),
            pltpu.VMEM((N, K), f32),
            pltpu.VMEM((80,), jnp.int32),
            pltpu.VMEM((K,), f32),
            pltpu.SemaphoreType.DMA,
        ],
    )
    return kern(Uh, Ul, Ypad, Eh, El, ETh, ETl, expE, Efull)


def kernel(inputs, Y, W, E):
    f32 = jnp.float32
    f64 = jnp.float64
    Wc = W[:, 1:K + 1]
    Wh = Wc.astype(f32)
    Wl = (Wc - Wh.astype(f64)).astype(f32)
    Ecore = E[1:-1, 1:-1]
    Eh = Ecore.astype(f32)
    El = (Ecore - Eh.astype(f64)).astype(f32)
    EcT = Ecore.T
    ETh = EcT.astype(f32)
    ETl = (EcT - ETh.astype(f64)).astype(f32)
    expE = jnp.exp(Ecore).astype(f32)
    Efull = E.astype(f32)

    x = inputs.reshape(_ROWS, M)
    Uh2, Ul2 = _u_pallas(x, Wh, Wl)
    Uh = Uh2.reshape(B, N + 1, K)
    Ul = Ul2.reshape(B, N + 1, K)

    Ypad = jnp.zeros((B, 80), jnp.int32).at[:, :N + 2].set(Y.astype(jnp.int32))

    yhat_pad, fpack = _sc_pallas(Uh, Ul, Ypad, Eh, El, ETh, ETl, expE, Efull)
    loss32 = _loss_pallas(fpack)

    Y_hat = yhat_pad[:, :N + 2]
    loss = loss32[0, 0].astype(f64)
    return (Y_hat, loss)
